# Initial kernel scaffold; baseline (speedup 1.0000x reference)
import jax
import jax.numpy as jnp
from jax.experimental import pallas as pl


def kernel(features, edge_index, W_proj, b_proj, betas, W_cls, b_cls):
    raise NotImplementedError("write your pallas kernel here")


# speedup vs baseline: 4.4703x; 4.4703x over previous
import functools

import jax
import jax.numpy as jnp
from jax import lax
from jax.experimental import pallas as pl
from jax.experimental.pallas import tpu as pltpu
from jax.experimental.pallas import tpu_sc as plsc

N_NODES = 10000
N_PAD = 10240
D = 256
DA = 272
E = 160000
E_PAD = 163840
CHUNK = 512
NBLK = 256
NWIN = N_PAD // NBLK
N_LAYER = 4
NCLS = 40

_SC_CORES = 2
_SC_SUBCORES = 16
_NW = _SC_CORES * _SC_SUBCORES
_B_PER_W = E_PAD // _NW
_GCHUNK = 128
_N_GCHUNK = _B_PER_W // _GCHUNK


def _proj_body(f_ref, wp_ref, bp_ref, x_ref):
    h = jnp.dot(f_ref[...], wp_ref[...], preferred_element_type=jnp.float32)
    x_ref[...] = jnp.maximum(h + bp_ref[...], 0.0)


def _proj(feats_p, W_proj, b_proj):
    return pl.pallas_call(
        _proj_body,
        grid=(NWIN,),
        in_specs=[
            pl.BlockSpec((NBLK, D), lambda i: (i, 0)),
            pl.BlockSpec((D, D), lambda i: (0, 0)),
            pl.BlockSpec((1, D), lambda i: (0, 0)),
        ],
        out_specs=pl.BlockSpec((NBLK, D), lambda i: (i, 0)),
        out_shape=jax.ShapeDtypeStruct((N_PAD, D), jnp.float32),
    )(feats_p, W_proj, b_proj)


def _renorm_body(a_ref, x_ref):
    a = a_ref[...]
    x_ref[...] = a[:, :D] / (a[:, D:D + 1] + 1e-12)


def _renorm(acc):
    return pl.pallas_call(
        _renorm_body,
        grid=(NWIN,),
        in_specs=[pl.BlockSpec((NBLK, DA), lambda i: (i, 0))],
        out_specs=pl.BlockSpec((NBLK, D), lambda i: (i, 0)),
        out_shape=jax.ShapeDtypeStruct((N_PAD, D), jnp.float32),
    )(acc)


def _cls_body(a_ref, wc_ref, bc_ref, o_ref):
    a = a_ref[...]
    h = a[:, :D] / (a[:, D:D + 1] + 1e-12)
    o_ref[...] = jnp.dot(h, wc_ref[...],
                         preferred_element_type=jnp.float32) + bc_ref[...]


def _cls(acc, W_cls, b_cls):
    return pl.pallas_call(
        _cls_body,
        grid=(NWIN,),
        in_specs=[
            pl.BlockSpec((NBLK, DA), lambda i: (i, 0)),
            pl.BlockSpec((D, NCLS), lambda i: (0, 0)),
            pl.BlockSpec((1, NCLS), lambda i: (0, 0)),
        ],
        out_specs=pl.BlockSpec((NBLK, NCLS), lambda i: (i, 0)),
        out_shape=jax.ShapeDtypeStruct((N_PAD, NCLS), jnp.float32),
    )(acc, W_cls, b_cls)


def _agg_body(beta_ref, dst_ref, g_ref, x_ref, acc_ref, vaug_ref):
    pid = pl.program_id(0)

    @pl.when(pid == 0)
    def _():
        acc_ref[...] = jnp.zeros_like(acc_ref)
        vaug_ref[:, D + 1:] = jnp.zeros((CHUNK, DA - D - 1), jnp.float32)

    beta = beta_ref[...]
    g = g_ref[...]
    ns = jnp.sqrt(jnp.sum(g * g, axis=1, keepdims=True))
    hn_src = g / (ns + 1e-12)
    dst = dst_ref[0]

    maxd = jnp.max(dst)
    w0 = jnp.min(dst) // NBLK

    def cond(w):
        return w * NBLK <= maxd

    def body(w):
        base = w * NBLK
        iota = lax.broadcasted_iota(jnp.int32, (NBLK, CHUNK), 0)
        T = ((dst - base) == iota).astype(jnp.float32)
        xwin = x_ref[pl.ds(base, NBLK), :]
        drow = lax.dot_general(T, xwin, (((0,), (0,)), ((), ())),
                               preferred_element_type=jnp.float32)
        nd = jnp.sqrt(jnp.sum(drow * drow, axis=1, keepdims=True))
        cos = jnp.sum(hn_src * drow, axis=1, keepdims=True) / (nd + 1e-12)
        ex = jnp.exp(beta * cos)
        vaug_ref[:, :D] = ex * g
        vaug_ref[:, D:D + 1] = ex
        upd = jnp.dot(T, vaug_ref[...], preferred_element_type=jnp.float32)
        acc_ref[pl.ds(base, NBLK), :] += upd
        nxt = jnp.min(jnp.where(dst >= base + NBLK, dst, jnp.int32(2 ** 30)))
        return nxt // NBLK

    lax.while_loop(cond, body, w0)


def _agg(beta, dstv, G, X):
    return pl.pallas_call(
        _agg_body,
        grid=(E_PAD // CHUNK,),
        in_specs=[
            pl.BlockSpec((1, 1), lambda i: (0, 0)),
            pl.BlockSpec((1, 1, CHUNK), lambda i: (i, 0, 0)),
            pl.BlockSpec((CHUNK, D), lambda i: (i, 0)),
            pl.BlockSpec((N_PAD, D), lambda i: (0, 0)),
        ],
        out_specs=pl.BlockSpec((N_PAD, DA), lambda i: (0, 0)),
        out_shape=jax.ShapeDtypeStruct((N_PAD, DA), jnp.float32),
        scratch_shapes=[pltpu.VMEM((CHUNK, DA), jnp.float32)],
    )(beta, dstv, G, X)


def _sc_gather_body(x_hbm, idx_hbm, out_hbm, idx_v, buf0, buf1, sem0, sem1):
    wid = lax.axis_index("s") * _SC_CORES + lax.axis_index("c")
    base = wid * _B_PER_W
    pltpu.sync_copy(idx_hbm.at[pl.ds(base, _B_PER_W)], idx_v)

    pltpu.async_copy(x_hbm.at[idx_v.at[pl.ds(0, _GCHUNK)]], buf0, sem0)

    @pl.loop(0, _N_GCHUNK, step=2)
    def _(i):
        off0 = pl.multiple_of(i * _GCHUNK, _GCHUNK)
        off1 = pl.multiple_of((i + 1) * _GCHUNK, _GCHUNK)
        off2 = pl.multiple_of((i + 2) * _GCHUNK, _GCHUNK)

        @pl.when(i + 1 < _N_GCHUNK)
        def _():
            pltpu.async_copy(x_hbm.at[idx_v.at[pl.ds(off1, _GCHUNK)]],
                             buf1, sem1)

        pltpu.make_async_copy(x_hbm.at[idx_v.at[pl.ds(off0, _GCHUNK)]],
                              buf0, sem0).wait()
        pltpu.sync_copy(buf0, out_hbm.at[pl.ds(base + off0, _GCHUNK)])

        @pl.when(i + 2 < _N_GCHUNK)
        def _():
            pltpu.async_copy(x_hbm.at[idx_v.at[pl.ds(off2, _GCHUNK)]],
                             buf0, sem0)

        @pl.when(i + 1 < _N_GCHUNK)
        def _():
            pltpu.make_async_copy(x_hbm.at[idx_v.at[pl.ds(off1, _GCHUNK)]],
                                  buf1, sem1).wait()
            pltpu.sync_copy(buf1, out_hbm.at[pl.ds(base + off1, _GCHUNK)])


@functools.lru_cache(maxsize=1)
def _make_sc_gather():
    return pl.kernel(
        _sc_gather_body,
        out_type=jax.ShapeDtypeStruct((E_PAD, D), jnp.float32),
        mesh=plsc.VectorSubcoreMesh(core_axis_name="c", subcore_axis_name="s"),
        scratch_types=[
            pltpu.VMEM((_B_PER_W,), jnp.int32),
            pltpu.VMEM((_GCHUNK, D), jnp.float32),
            pltpu.VMEM((_GCHUNK, D), jnp.float32),
            pltpu.SemaphoreType.DMA,
            pltpu.SemaphoreType.DMA,
        ],
    )


def _sc_gather(X, idx):
    return _make_sc_gather()(X, idx)


def kernel(features, edge_index, W_proj, b_proj, betas, W_cls, b_cls):
    src = edge_index[0]
    dst = edge_index[1]
    order = jnp.argsort(dst)
    src_s = jnp.concatenate(
        [src[order], jnp.zeros((E_PAD - E,), jnp.int32)])
    dst_s = jnp.concatenate(
        [dst[order], jnp.full((E_PAD - E,), N_PAD - 1, jnp.int32)])
    dstv = dst_s.reshape(E_PAD // CHUNK, 1, CHUNK)
    feats_p = jnp.pad(features, ((0, N_PAD - N_NODES), (0, 0)))

    X = _proj(feats_p, W_proj, b_proj.reshape(1, D))
    acc = None
    for i in range(N_LAYER):
        if i > 0:
            X = _renorm(acc)
        G = _sc_gather(X, src_s)
        acc = _agg(betas[i].reshape(1, 1), dstv, G, X)
    logits = _cls(acc, W_cls, b_cls.reshape(1, NCLS))
    return logits[:N_NODES]


# speedup vs baseline: 5.8033x; 1.2982x over previous
import functools

import jax
import jax.numpy as jnp
from jax import lax
from jax.experimental import pallas as pl
from jax.experimental.pallas import tpu as pltpu
from jax.experimental.pallas import tpu_sc as plsc

N_NODES = 10000
N_PAD = 10240
D = 256
DA = 272
E = 160000
E_PAD = 163840
CHUNK = 512
NBLK = 256
NWIN = N_PAD // NBLK
N_LAYER = 4
NCLS = 40

_SC_CORES = 2
_SC_SUBCORES = 16
_NW = _SC_CORES * _SC_SUBCORES
_B_PER_W = E_PAD // _NW
_GCHUNK = 128
_N_GCHUNK = _B_PER_W // _GCHUNK


def _proj_body(f_ref, wp_ref, bp_ref, x_ref):
    h = jnp.dot(f_ref[...], wp_ref[...], preferred_element_type=jnp.float32)
    x_ref[...] = jnp.maximum(h + bp_ref[...], 0.0)


def _proj(feats_p, W_proj, b_proj):
    return pl.pallas_call(
        _proj_body,
        grid=(NWIN,),
        in_specs=[
            pl.BlockSpec((NBLK, D), lambda i: (i, 0)),
            pl.BlockSpec((D, D), lambda i: (0, 0)),
            pl.BlockSpec((1, D), lambda i: (0, 0)),
        ],
        out_specs=pl.BlockSpec((NBLK, D), lambda i: (i, 0)),
        out_shape=jax.ShapeDtypeStruct((N_PAD, D), jnp.float32),
    )(feats_p, W_proj, b_proj)


def _renorm_body(a_ref, x_ref):
    a = a_ref[...]
    x_ref[...] = a[:, :D] / (a[:, D:D + 1] + 1e-12)


def _renorm(acc):
    return pl.pallas_call(
        _renorm_body,
        grid=(NWIN,),
        in_specs=[pl.BlockSpec((NBLK, DA), lambda i: (i, 0))],
        out_specs=pl.BlockSpec((NBLK, D), lambda i: (i, 0)),
        out_shape=jax.ShapeDtypeStruct((N_PAD, D), jnp.float32),
    )(acc)


def _cls_body(a_ref, wc_ref, bc_ref, o_ref):
    a = a_ref[...]
    h = a[:, :D] / (a[:, D:D + 1] + 1e-12)
    o_ref[...] = jnp.dot(h, wc_ref[...],
                         preferred_element_type=jnp.float32) + bc_ref[...]


def _cls(acc, W_cls, b_cls):
    return pl.pallas_call(
        _cls_body,
        grid=(NWIN,),
        in_specs=[
            pl.BlockSpec((NBLK, DA), lambda i: (i, 0)),
            pl.BlockSpec((D, NCLS), lambda i: (0, 0)),
            pl.BlockSpec((1, NCLS), lambda i: (0, 0)),
        ],
        out_specs=pl.BlockSpec((NBLK, NCLS), lambda i: (i, 0)),
        out_shape=jax.ShapeDtypeStruct((N_PAD, NCLS), jnp.float32),
    )(acc, W_cls, b_cls)


def _agg_body(beta_ref, dst_ref, g_ref, x_ref, acc_in_ref, acc_ref, vaug_ref):
    pid = pl.program_id(0)

    @pl.when(pid == 0)
    def _():
        acc_ref[...] = acc_in_ref[...]
        vaug_ref[:, D + 1:] = jnp.zeros((CHUNK, DA - D - 1), jnp.float32)

    beta = beta_ref[...]
    g = g_ref[...]
    ns = jnp.sqrt(jnp.sum(g * g, axis=1, keepdims=True))
    hn_src = g / (ns + 1e-12)
    dst = dst_ref[0]

    maxd = jnp.max(dst)
    w0 = jnp.min(dst) // NBLK

    def cond(w):
        return w * NBLK <= maxd

    def body(w):
        base = w * NBLK
        iota = lax.broadcasted_iota(jnp.int32, (NBLK, CHUNK), 0)
        T = ((dst - base) == iota).astype(jnp.float32)
        xwin = x_ref[pl.ds(base, NBLK), :]
        drow = lax.dot_general(T, xwin, (((0,), (0,)), ((), ())),
                               preferred_element_type=jnp.float32)
        nd = jnp.sqrt(jnp.sum(drow * drow, axis=1, keepdims=True))
        cos = jnp.sum(hn_src * drow, axis=1, keepdims=True) / (nd + 1e-12)
        ex = jnp.exp(beta * cos)
        vaug_ref[:, :D] = ex * g
        vaug_ref[:, D:D + 1] = ex
        upd = jnp.dot(T, vaug_ref[...], preferred_element_type=jnp.float32)
        acc_ref[pl.ds(base, NBLK), :] += upd
        nxt = jnp.min(jnp.where(dst >= base + NBLK, dst, jnp.int32(2 ** 30)))
        return nxt // NBLK

    lax.while_loop(cond, body, w0)


def _agg(beta, dstv, G, X, acc_in):
    n_chunks = G.shape[0] // CHUNK
    return pl.pallas_call(
        _agg_body,
        grid=(n_chunks,),
        in_specs=[
            pl.BlockSpec((1, 1), lambda i: (0, 0)),
            pl.BlockSpec((1, 1, CHUNK), lambda i: (i, 0, 0)),
            pl.BlockSpec((CHUNK, D), lambda i: (i, 0)),
            pl.BlockSpec((N_PAD, D), lambda i: (0, 0)),
            pl.BlockSpec((N_PAD, DA), lambda i: (0, 0)),
        ],
        out_specs=pl.BlockSpec((N_PAD, DA), lambda i: (0, 0)),
        out_shape=jax.ShapeDtypeStruct((N_PAD, DA), jnp.float32),
        scratch_shapes=[pltpu.VMEM((CHUNK, DA), jnp.float32)],
    )(beta, dstv, G, X, acc_in)


@functools.lru_cache(maxsize=4)
def _make_sc_gather(n_idx):
    b_per_w = n_idx // _NW
    n_gchunk = b_per_w // _GCHUNK

    def body(x_hbm, idx_hbm, out_hbm, idx_v, buf0, buf1, sem0, sem1):
        wid = lax.axis_index("s") * _SC_CORES + lax.axis_index("c")
        base = wid * b_per_w
        pltpu.sync_copy(idx_hbm.at[pl.ds(base, b_per_w)], idx_v)

        pltpu.async_copy(x_hbm.at[idx_v.at[pl.ds(0, _GCHUNK)]], buf0, sem0)

        @pl.loop(0, n_gchunk, step=2)
        def _(i):
            off0 = pl.multiple_of(i * _GCHUNK, _GCHUNK)
            off1 = pl.multiple_of((i + 1) * _GCHUNK, _GCHUNK)
            off2 = pl.multiple_of((i + 2) * _GCHUNK, _GCHUNK)

            @pl.when(i + 1 < n_gchunk)
            def _():
                pltpu.async_copy(x_hbm.at[idx_v.at[pl.ds(off1, _GCHUNK)]],
                                 buf1, sem1)

            pltpu.make_async_copy(x_hbm.at[idx_v.at[pl.ds(off0, _GCHUNK)]],
                                  buf0, sem0).wait()
            pltpu.sync_copy(buf0, out_hbm.at[pl.ds(base + off0, _GCHUNK)])

            @pl.when(i + 2 < n_gchunk)
            def _():
                pltpu.async_copy(x_hbm.at[idx_v.at[pl.ds(off2, _GCHUNK)]],
                                 buf0, sem0)

            @pl.when(i + 1 < n_gchunk)
            def _():
                pltpu.make_async_copy(x_hbm.at[idx_v.at[pl.ds(off1, _GCHUNK)]],
                                      buf1, sem1).wait()
                pltpu.sync_copy(buf1, out_hbm.at[pl.ds(base + off1, _GCHUNK)])

    return pl.kernel(
        body,
        out_type=jax.ShapeDtypeStruct((n_idx, D), jnp.float32),
        mesh=plsc.VectorSubcoreMesh(core_axis_name="c", subcore_axis_name="s"),
        scratch_types=[
            pltpu.VMEM((b_per_w,), jnp.int32),
            pltpu.VMEM((_GCHUNK, D), jnp.float32),
            pltpu.VMEM((_GCHUNK, D), jnp.float32),
            pltpu.SemaphoreType.DMA,
            pltpu.SemaphoreType.DMA,
        ],
    )


def _sc_gather(X, idx):
    return _make_sc_gather(idx.shape[0])(X, idx)


def kernel(features, edge_index, W_proj, b_proj, betas, W_cls, b_cls):
    src = edge_index[0]
    dst = edge_index[1]
    order = jnp.argsort(dst)
    src_s = jnp.concatenate(
        [src[order], jnp.zeros((E_PAD - E,), jnp.int32)])
    dst_s = jnp.concatenate(
        [dst[order], jnp.full((E_PAD - E,), N_PAD - 1, jnp.int32)])
    dstv = dst_s.reshape(E_PAD // CHUNK, 1, CHUNK)
    feats_p = jnp.pad(features, ((0, N_PAD - N_NODES), (0, 0)))

    half = E_PAD // 2
    src_a, src_b = src_s[:half], src_s[half:]
    dstv_a, dstv_b = dstv[: half // CHUNK], dstv[half // CHUNK:]

    X = _proj(feats_p, W_proj, b_proj.reshape(1, D))
    acc0 = jnp.zeros((N_PAD, DA), jnp.float32)
    acc = None
    for i in range(N_LAYER):
        if i > 0:
            X = _renorm(acc)
        beta = betas[i].reshape(1, 1)
        G_a = _sc_gather(X, src_a)
        G_b = _sc_gather(X, src_b)
        acc = _agg(beta, dstv_a, G_a, X, acc0)
        acc = _agg(beta, dstv_b, G_b, X, acc)
    logits = _cls(acc, W_cls, b_cls.reshape(1, NCLS))
    return logits[:N_NODES]
